# baseline (speedup 1.0000x reference)
import functools
import math
import random

import numpy as np
import jax
import jax.numpy as jnp
from jax import lax
from jax.experimental import pallas as pl
from jax.experimental.pallas import tpu as pltpu
from jax.experimental.pallas import tpu_sc as plsc


@functools.lru_cache(maxsize=None)
def _shuffle_tables(T, B):
    n = int(math.sqrt(T))

    def one(rand):
        fi = np.arange(T).reshape(n, n)
        if rand == 0:
            a = fi[:, 0::2].copy()
            b = fi[:, 1::2].copy()
        else:
            a = fi[:, 1::2].copy()
            b = fi[:, 0::2].copy()
        for i in range(16):
            if i % 2 != 0:
                tmp = a[i].copy()
                a[i] = b[i]
                b[i] = tmp
        fwd = np.concatenate((a, b)).reshape(-1)
        return fwd, np.argsort(fwd)

    rng = random.Random(0)
    pairs = [one(rng.randint(0, 1)) for _ in range(B)]
    fwd = np.stack([p[0] for p in pairs], axis=-1).astype(np.int32)
    bwd = np.stack([p[1] for p in pairs], axis=-1).astype(np.int32)
    return fwd, bwd


@functools.lru_cache(maxsize=None)
def _build_gather(R, C, n_rows):
    info = plsc.get_sparse_core_info()
    num_workers = info.num_cores * info.num_subcores
    rows_w = R // num_workers
    chunk = 16
    nbuf = 8
    n_chunks = rows_w // chunk
    assert rows_w % chunk == 0 and n_chunks >= nbuf

    mesh = plsc.VectorSubcoreMesh(core_axis_name="c", subcore_axis_name="s")

    @functools.partial(
        pl.kernel,
        mesh=mesh,
        out_type=jax.ShapeDtypeStruct((R, C), jnp.float32),
        scratch_types=(
            [pltpu.VMEM((rows_w,), jnp.int32)]
            + [pltpu.VMEM((chunk, C), jnp.float32) for _ in range(nbuf)]
            + [pltpu.SemaphoreType.DMA for _ in range(2 * nbuf)]
        ),
    )
    def gather_rows(flat_hbm, gidx_hbm, out_hbm, idx_v, *bufs_and_sems):
        bufs = bufs_and_sems[:nbuf]
        gsem = bufs_and_sems[nbuf:2 * nbuf]
        ssem = bufs_and_sems[2 * nbuf:]
        wid = lax.axis_index("s") * info.num_cores + lax.axis_index("c")
        base = wid * rows_w
        pltpu.sync_copy(gidx_hbm.at[pl.ds(base, rows_w)], idx_v)

        def gather_desc(c, j):
            return pltpu.make_async_copy(
                flat_hbm.at[idx_v.at[pl.ds(c * chunk, chunk)]],
                bufs[j], gsem[j])

        def scatter_desc(c, j):
            return pltpu.make_async_copy(
                bufs[j], out_hbm.at[pl.ds(base + c * chunk, chunk)], ssem[j])

        n_groups = n_chunks // nbuf
        assert n_chunks % nbuf == 0 and n_groups >= 2


        for j in range(nbuf - 1):
            gather_desc(j, j).start()
        gather_desc(0, 0).wait()
        gather_desc(nbuf - 1, nbuf - 1).start()
        scatter_desc(0, 0).start()
        for j in range(1, nbuf):
            gather_desc(j, j).wait()
            scatter_desc(j - 1, j - 1).wait()
            gather_desc(j + nbuf - 1, j - 1).start()
            scatter_desc(j, j).start()

        def group_body(g, carry):
            c0 = g * nbuf
            for j in range(nbuf):
                c = c0 + j
                gather_desc(c, j).wait()
                scatter_desc(c - 1, (j - 1) % nbuf).wait()
                gather_desc(c + nbuf - 1, (j - 1) % nbuf).start()
                scatter_desc(c, j).start()
            return carry

        if n_groups > 2:
            lax.fori_loop(1, n_groups - 1, group_body, 0, unroll=False)

        c0 = (n_groups - 1) * nbuf
        for j in range(nbuf):
            c = c0 + j
            gather_desc(c, j).wait()
            scatter_desc(c - 1, (j - 1) % nbuf).wait()
            if c + nbuf - 1 < n_chunks:
                gather_desc(c + nbuf - 1, (j - 1) % nbuf).start()
            scatter_desc(c, j).start()
        scatter_desc(n_chunks - 1, nbuf - 1).wait()

    return gather_rows


@functools.lru_cache(maxsize=None)
def _build_select_tc(remain_T, B, C):
    def t_of(p, k):
        return 16 * (k // 8) + 8 * p + (k % 8)

    def body(pairs_ref, w_ref, out_ref):
        x0 = pairs_ref[0, 0]
        x1 = pairs_ref[0, 1]
        w = w_ref[0]
        out_ref[0] = x0 + w * (x1 - x0)

    return pl.pallas_call(
        body,
        grid=(2, remain_T // 2),
        in_specs=[
            pl.BlockSpec((1, 2, B, C), lambda p, k: (t_of(p, k), 0, 0, 0)),
            pl.BlockSpec((1, B, C), lambda p, k: (p, 0, 0)),
        ],
        out_specs=pl.BlockSpec((1, B, C), lambda p, k: (t_of(p, k), 0, 0)),
        out_shape=jax.ShapeDtypeStruct((remain_T, B, C), jnp.float32),
    )


def kernel(patches):
    T, B, C = patches.shape
    remain_T = T // 2
    fwd_np, bwd_np = _shuffle_tables(T, B)
    s0 = (fwd_np[0] & 1).astype(np.float32)
    w_np = np.stack([s0, 1.0 - s0], axis=0)
    w_full = np.broadcast_to(w_np[:, :, None], (2, B, C)).copy()
    pairs = patches.reshape(remain_T, 2, B, C)
    shuffled = _build_select_tc(remain_T, B, C)(pairs, jnp.asarray(w_full))
    return (shuffled,
            jnp.asarray(fwd_np, dtype=jnp.int32),
            jnp.asarray(bwd_np, dtype=jnp.int32))


# speedup vs baseline: 1.4963x; 1.4963x over previous
import functools
import math
import random

import numpy as np
import jax
import jax.numpy as jnp
from jax import lax
from jax.experimental import pallas as pl
from jax.experimental.pallas import tpu as pltpu
from jax.experimental.pallas import tpu_sc as plsc


@functools.lru_cache(maxsize=None)
def _shuffle_tables(T, B):
    n = int(math.sqrt(T))

    def one(rand):
        fi = np.arange(T).reshape(n, n)
        if rand == 0:
            a = fi[:, 0::2].copy()
            b = fi[:, 1::2].copy()
        else:
            a = fi[:, 1::2].copy()
            b = fi[:, 0::2].copy()
        for i in range(16):
            if i % 2 != 0:
                tmp = a[i].copy()
                a[i] = b[i]
                b[i] = tmp
        fwd = np.concatenate((a, b)).reshape(-1)
        return fwd, np.argsort(fwd)

    rng = random.Random(0)
    pairs = [one(rng.randint(0, 1)) for _ in range(B)]
    fwd = np.stack([p[0] for p in pairs], axis=-1).astype(np.int32)
    bwd = np.stack([p[1] for p in pairs], axis=-1).astype(np.int32)
    return fwd, bwd


@functools.lru_cache(maxsize=None)
def _build_gather(R, C, n_rows):
    info = plsc.get_sparse_core_info()
    num_workers = info.num_cores * info.num_subcores
    rows_w = R // num_workers
    chunk = 16
    nbuf = 8
    n_chunks = rows_w // chunk
    assert rows_w % chunk == 0 and n_chunks >= nbuf

    mesh = plsc.VectorSubcoreMesh(core_axis_name="c", subcore_axis_name="s")

    @functools.partial(
        pl.kernel,
        mesh=mesh,
        out_type=jax.ShapeDtypeStruct((R, C), jnp.float32),
        scratch_types=(
            [pltpu.VMEM((rows_w,), jnp.int32)]
            + [pltpu.VMEM((chunk, C), jnp.float32) for _ in range(nbuf)]
            + [pltpu.SemaphoreType.DMA for _ in range(2 * nbuf)]
        ),
    )
    def gather_rows(flat_hbm, gidx_hbm, out_hbm, idx_v, *bufs_and_sems):
        bufs = bufs_and_sems[:nbuf]
        gsem = bufs_and_sems[nbuf:2 * nbuf]
        ssem = bufs_and_sems[2 * nbuf:]
        wid = lax.axis_index("s") * info.num_cores + lax.axis_index("c")
        base = wid * rows_w
        pltpu.sync_copy(gidx_hbm.at[pl.ds(base, rows_w)], idx_v)

        def gather_desc(c, j):
            return pltpu.make_async_copy(
                flat_hbm.at[idx_v.at[pl.ds(c * chunk, chunk)]],
                bufs[j], gsem[j])

        def scatter_desc(c, j):
            return pltpu.make_async_copy(
                bufs[j], out_hbm.at[pl.ds(base + c * chunk, chunk)], ssem[j])

        n_groups = n_chunks // nbuf
        assert n_chunks % nbuf == 0 and n_groups >= 2


        for j in range(nbuf - 1):
            gather_desc(j, j).start()
        gather_desc(0, 0).wait()
        gather_desc(nbuf - 1, nbuf - 1).start()
        scatter_desc(0, 0).start()
        for j in range(1, nbuf):
            gather_desc(j, j).wait()
            scatter_desc(j, j).start()
            scatter_desc(j - 1, j - 1).wait()
            gather_desc(j + nbuf - 1, j - 1).start()

        def group_body(g, carry):
            c0 = g * nbuf
            for j in range(nbuf):
                c = c0 + j
                gather_desc(c, j).wait()
                scatter_desc(c, j).start()
                scatter_desc(c - 1, (j - 1) % nbuf).wait()
                gather_desc(c + nbuf - 1, (j - 1) % nbuf).start()
            return carry

        if n_groups > 2:
            lax.fori_loop(1, n_groups - 1, group_body, 0, unroll=False)

        c0 = (n_groups - 1) * nbuf
        for j in range(nbuf):
            c = c0 + j
            gather_desc(c, j).wait()
            scatter_desc(c, j).start()
            scatter_desc(c - 1, (j - 1) % nbuf).wait()
            if c + nbuf - 1 < n_chunks:
                gather_desc(c + nbuf - 1, (j - 1) % nbuf).start()
        scatter_desc(n_chunks - 1, nbuf - 1).wait()

    return gather_rows


def kernel(patches):
    T, B, C = patches.shape
    remain_T = T // 2
    fwd_np, bwd_np = _shuffle_tables(T, B)
    gidx_np = (fwd_np[:remain_T].astype(np.int64) * B
               + np.arange(B, dtype=np.int64)[None, :]).reshape(-1)
    gidx = jnp.asarray(gidx_np.astype(np.int32))
    flat = patches.reshape(T * B, C)
    R = remain_T * B
    out = _build_gather(R, C, T * B)(flat, gidx)
    shuffled = out.reshape(remain_T, B, C)
    return (shuffled,
            jnp.asarray(fwd_np, dtype=jnp.int32),
            jnp.asarray(bwd_np, dtype=jnp.int32))
